# baseline (speedup 1.0000x reference)
import jax
import jax.numpy as jnp
from jax import lax
from jax.experimental import pallas as pl
from jax.experimental.pallas import tpu as pltpu
from jax.experimental.pallas import tpu_sc as plsc

N = 50000
E = 800000
D = 64
L = 3

NC = 2
NS = 16
NW = NC * NS
LN = 16

SPAN = 2048
SUB = 512
NSUB = SPAN // SUB
NP = NW * SPAN
STG = 4160
FL = 2048
CHA = 2000
CHA2 = 2048
CAP = E + 2 * FL
CEB = 128
DP = 128
CEC = 192
EWC = 192 * 131
EPAD = NW * EWC

_mesh = plsc.VectorSubcoreMesh(core_axis_name="c", subcore_axis_name="s")
_params = pltpu.CompilerParams(needs_layout_passes=False)
_params_nt = pltpu.CompilerParams(needs_layout_passes=False,
                                  use_tc_tiling_on_sc=False)


def _wid():
    return lax.axis_index("s") * NC + lax.axis_index("c")


def _iota():
    return lax.iota(jnp.int32, LN)


def _s0(v):
    return v[0]


def _shift_down(scr, v, k):
    scr[...] = v
    return plsc.load_gather(scr, [jnp.maximum(_iota() - k, 0)])


def _compact_store(stg, off, pk, m):
    key = jnp.where(m, _iota(), LN + _iota())
    _, spk = plsc.sort_key_val(key, pk)
    stg[pl.ds(off, LN)] = spk
    return off + _s0(plsc.all_reduce_population_count(m))


def _compact_store_vec(stg, offv, pk, m):
    iota = _iota()
    mi = m.astype(jnp.int32)
    p = mi
    for k in (1, 2, 4, 8):
        shifted = p[jnp.maximum(iota - k, 0)]
        p = p + jnp.where(iota >= k, shifted, 0)
    plsc.store_scatter(stg, [offv + p - mi], pk, mask=m)
    return offv + p[jnp.full((LN,), LN - 1, jnp.int32)]


def _part_body(src_hbm, dst_hbm, pk_hbm, cnt_hbm, src_v, dst_v, stg_v, cnt_v):
    w = _wid()
    lo = w * SPAN

    def zi(i, c):
        stg_v[pl.ds(i * LN, LN)] = jnp.zeros((LN,), jnp.int32)
        return c

    lax.fori_loop(0, STG // LN, zi, 0)

    def chunk(ci, carry):
        offv0, flushed = carry
        pltpu.sync_copy(src_hbm.at[pl.ds(pl.multiple_of(ci * CHA, 8), CHA)],
                        src_v)
        pltpu.sync_copy(dst_hbm.at[pl.ds(pl.multiple_of(ci * CHA, 8), CHA)],
                        dst_v)

        def vstep(j, offv):
            d = dst_v[pl.ds(j * LN, LN)]
            s = src_v[pl.ds(j * LN, LN)]
            dl = d - lo
            m = (dl >= 0) & (dl < SPAN)
            pk = jnp.bitwise_or(jnp.left_shift(dl, 16), s)
            return _compact_store_vec(stg_v, offv, pk, m)

        offv = lax.fori_loop(0, CHA // LN, vstep, offv0)
        off = _s0(offv)
        do = off >= FL

        @pl.when(do)
        def _flush():
            pltpu.sync_copy(
                stg_v.at[pl.ds(0, FL)],
                pk_hbm.at[pl.ds(pl.multiple_of(w * CAP + flushed, 8), FL)])
            nt = (off - FL + LN - 1) >> 4

            def mv(i, c):
                stg_v[pl.ds(i * LN, LN)] = stg_v[pl.ds(FL + i * LN, LN)]
                return c

            lax.fori_loop(0, nt, mv, 0)

        offv = jnp.where(do, offv - FL, offv)
        flushed = jnp.where(do, flushed + FL, flushed)
        return offv, flushed

    offv, flushed = lax.fori_loop(
        0, E // CHA, chunk,
        (jnp.zeros((LN,), jnp.int32), jnp.int32(0)))
    pltpu.sync_copy(stg_v.at[pl.ds(0, FL)],
                    pk_hbm.at[pl.ds(pl.multiple_of(w * CAP + flushed, 8), FL)])
    cnt_v[...] = offv + flushed
    pltpu.sync_copy(cnt_v, cnt_hbm.at[pl.ds(pl.multiple_of(w * LN, 8), LN)])


_partition = pl.kernel(
    _part_body,
    out_type=(jax.ShapeDtypeStruct((NW * CAP,), jnp.int32),
              jax.ShapeDtypeStruct((NW * LN,), jnp.int32)),
    mesh=_mesh,
    compiler_params=_params,
    scratch_types=[
        pltpu.VMEM((CHA,), jnp.int32),
        pltpu.VMEM((CHA,), jnp.int32),
        pltpu.VMEM((STG,), jnp.int32),
        pltpu.VMEM((LN,), jnp.int32),
    ],
)


def _split_body(pk_hbm, cnt_hbm, pk2_hbm, cnt2_hbm, pkv, stg_v, cnt_v):
    w = _wid()
    iota = _iota()
    pltpu.sync_copy(cnt_hbm.at[pl.ds(pl.multiple_of(w * LN, 8), LN)], cnt_v)
    cnt = _s0(cnt_v[...])

    def zi(i, c):
        stg_v[pl.ds(i * LN, LN)] = jnp.zeros((LN,), jnp.int32)
        return c

    lax.fori_loop(0, NSUB * STG // LN, zi, 0)

    nch = (cnt + (CHA2 - 1)) >> 11

    def chunk(ci, carry):
        pltpu.sync_copy(
            pk_hbm.at[pl.ds(pl.multiple_of(w * CAP + ci * CHA2, 8), CHA2)],
            pkv)

        def vstep(j, offs):
            pk = pkv[pl.ds(j * LN, LN)]
            valid = (ci * CHA2 + j * LN + iota) < cnt
            sub = jnp.right_shift(pk, 16 + 9)
            new = []
            for p in range(NSUB):
                m = valid & (sub == p)
                new.append(_compact_store_vec(
                    stg_v.at[pl.ds(p * STG, STG)], offs[p], pk, m))
            return tuple(new)

        offvs = lax.fori_loop(0, CHA2 // LN, vstep, carry[0])
        new_offs, new_fl = [], []
        for p in range(NSUB):
            off = _s0(offvs[p])
            flushed = carry[1][p]
            do = off >= FL

            @pl.when(do)
            def _flush(p=p, flushed=flushed, off=off):
                base = (w * NSUB + p) * CAP + flushed
                pltpu.sync_copy(
                    stg_v.at[pl.ds(p * STG, FL)],
                    pk2_hbm.at[pl.ds(pl.multiple_of(base, 8), FL)])
                nt = (off - FL + LN - 1) >> 4

                def mv(i, c, p=p):
                    stg_v[pl.ds(p * STG + i * LN, LN)] = (
                        stg_v[pl.ds(p * STG + FL + i * LN, LN)])
                    return c

                lax.fori_loop(0, nt, mv, 0)

            new_offs.append(jnp.where(do, offvs[p] - FL, offvs[p]))
            new_fl.append(jnp.where(do, flushed + FL, flushed))
        return tuple(new_offs), tuple(new_fl)

    offvs, flushes = lax.fori_loop(
        0, nch, chunk,
        ((jnp.zeros((LN,), jnp.int32),) * NSUB, (jnp.int32(0),) * NSUB))
    for p in range(NSUB):
        flushed = flushes[p]
        base = (w * NSUB + p) * CAP + flushed
        pltpu.sync_copy(stg_v.at[pl.ds(p * STG, FL)],
                        pk2_hbm.at[pl.ds(pl.multiple_of(base, 8), FL)])
        cnt_v[...] = offvs[p] + flushed
        pltpu.sync_copy(
            cnt_v,
            cnt2_hbm.at[pl.ds(pl.multiple_of((w * NSUB + p) * LN, 8), LN)])


_split = pl.kernel(
    _split_body,
    out_type=(jax.ShapeDtypeStruct((NW * NSUB * CAP,), jnp.int32),
              jax.ShapeDtypeStruct((NW * NSUB * LN,), jnp.int32)),
    mesh=_mesh,
    compiler_params=_params,
    scratch_types=[
        pltpu.VMEM((CHA2,), jnp.int32),
        pltpu.VMEM((NSUB * STG,), jnp.int32),
        pltpu.VMEM((LN,), jnp.int32),
    ],
)


def _segmin_body(xw_hbm, pk2_hbm, cnt2_hbm, agg_hbm,
                 pkv0, pkv1, idxv0, idxv1, dlv0, dlv1, rows0, rows1, agg,
                 cntv, sem0, sem1):
    w = _wid()
    iota = _iota()
    inf_v = jnp.full((LN,), jnp.inf, jnp.float32)
    pkv = (pkv0, pkv1)
    idxv = (idxv0, idxv1)
    dlv = (dlv0, dlv1)
    rows = (rows0, rows1)
    sem = (sem0, sem1)

    for p in range(NSUB):
        pltpu.sync_copy(
            cnt2_hbm.at[pl.ds(pl.multiple_of((w * NSUB + p) * LN, 8), LN)],
            cntv)
        cnt = _s0(cntv[...])

        def zrow(r, c):
            for cc in range(D // LN):
                agg[r, pl.ds(cc * LN, LN)] = inf_v
            return c

        lax.fori_loop(0, SUB, zrow, 0)

        nch = (cnt + (CEB - 1)) >> 7
        lbase = (w * NSUB + p) * CAP

        def issue(ci, par, lbase=lbase):
            pltpu.sync_copy(
                pk2_hbm.at[pl.ds(pl.multiple_of(lbase + ci * CEB, 8), CEB)],
                pkv[par])
            for j in range(CEB // LN):
                pk = pkv[par][pl.ds(j * LN, LN)]
                idxv[par][pl.ds(j * LN, LN)] = pk & 0xFFFF
                dlv[par][pl.ds(j * LN, LN)] = (
                    jnp.right_shift(pk, 16) & (SUB - 1))
            pltpu.async_copy(xw_hbm.at[idxv[par]], rows[par], sem[par])

        def compute(ci, par, cnt=cnt):
            pltpu.make_async_copy(xw_hbm.at[idxv[par]], rows[par],
                                  sem[par]).wait()

            def batch(j, c1):
                dl = dlv[par][pl.ds(j * LN, LN)]
                valid = (ci * CEB + j * LN + iota) < cnt
                rowi = j * LN + iota
                for k in range(D):
                    cvec = (iota + k) & (D - 1)
                    rv = plsc.load_gather(rows[par], [rowi, cvec])
                    av = plsc.load_gather(agg, [dl, cvec])
                    plsc.store_scatter(agg, [dl, cvec],
                                       jnp.minimum(av, rv), mask=valid)
                return c1

            lax.fori_loop(0, CEB // LN, batch, 0)

        @pl.when(nch > 0)
        def _prime():
            issue(0, 0)

        def chunk2(cj, c0):
            ci0 = cj * 2
            ci1 = ci0 + 1

            @pl.when(ci1 < nch)
            def _i1():
                issue(ci1, 1)

            compute(ci0, 0)

            @pl.when(ci0 + 2 < nch)
            def _i2():
                issue(ci0 + 2, 0)

            @pl.when(ci1 < nch)
            def _c1():
                compute(ci1, 1)

            return c0

        lax.fori_loop(0, (nch + 1) >> 1, chunk2, 0)
        pltpu.sync_copy(agg, agg_hbm.at[pl.ds(w * SPAN + p * SUB, SUB)])


_segmin = pl.kernel(
    _segmin_body,
    out_type=jax.ShapeDtypeStruct((NP, D), jnp.float32),
    mesh=_mesh,
    compiler_params=_params_nt,
    scratch_types=[
        pltpu.VMEM((CEB,), jnp.int32),
        pltpu.VMEM((CEB,), jnp.int32),
        pltpu.VMEM((CEB,), jnp.int32),
        pltpu.VMEM((CEB,), jnp.int32),
        pltpu.VMEM((CEB,), jnp.int32),
        pltpu.VMEM((CEB,), jnp.int32),
        pltpu.VMEM((CEB, D), jnp.float32),
        pltpu.VMEM((CEB, D), jnp.float32),
        pltpu.VMEM((SUB, D), jnp.float32),
        pltpu.VMEM((LN,), jnp.int32),
        pltpu.SemaphoreType.DMA,
        pltpu.SemaphoreType.DMA,
    ],
)


def _readout_body(xa_hbm, xb_hbm, src_hbm, dst_hbm, tgt_hbm, wv_hbm, bb_hbm,
                  out_hbm, srcv0, srcv1, dstv0, dstv1, tgtv0, tgtv1,
                  rowsa0, rowsa1, rowsb0, rowsb1, wvv, bbv, accv,
                  sem0, sem1):
    w = _wid()
    iota = _iota()
    pltpu.sync_copy(wv_hbm, wvv)
    pltpu.sync_copy(bb_hbm, bbv)
    b_splat = bbv[...]
    base = w * EWC
    srcv = (srcv0, srcv1)
    dstv = (dstv0, dstv1)
    tgtv = (tgtv0, tgtv1)
    rowsa = (rowsa0, rowsa1)
    rowsb = (rowsb0, rowsb1)
    sem = (sem0, sem1)
    nch = EWC // CEC

    def issue(ci, par):
        off = base + ci * CEC
        pltpu.sync_copy(src_hbm.at[pl.ds(pl.multiple_of(off, 8), CEC)],
                        srcv[par])
        pltpu.sync_copy(dst_hbm.at[pl.ds(pl.multiple_of(off, 8), CEC)],
                        dstv[par])
        pltpu.sync_copy(tgt_hbm.at[pl.ds(pl.multiple_of(off, 8), CEC)],
                        tgtv[par])
        pltpu.async_copy(xa_hbm.at[srcv[par].at[pl.ds(0, 96)]],
                         rowsa[par].at[pl.ds(0, 96)], sem[par])
        pltpu.async_copy(xa_hbm.at[srcv[par].at[pl.ds(96, 96)]],
                         rowsa[par].at[pl.ds(96, 96)], sem[par])
        pltpu.async_copy(xb_hbm.at[dstv[par].at[pl.ds(0, 96)]],
                         rowsb[par].at[pl.ds(0, 96)], sem[par])
        pltpu.async_copy(xb_hbm.at[dstv[par].at[pl.ds(96, 96)]],
                         rowsb[par].at[pl.ds(96, 96)], sem[par])

    def compute(ci, par, acc):
        off = base + ci * CEC
        pltpu.make_async_copy(xa_hbm.at[srcv[par]], rowsa[par],
                              sem[par]).wait()
        pltpu.make_async_copy(xb_hbm.at[dstv[par]], rowsb[par],
                              sem[par]).wait()

        def batch(j, acc2):
            rowi = j * LN + iota
            m = (off + rowi) < E
            y = jnp.zeros((LN,), jnp.float32)
            for k in range(D):
                cvec = (iota + k) & (D - 1)
                va = plsc.load_gather(rowsa[par], [rowi, cvec])
                vb = plsc.load_gather(rowsb[par], [rowi, cvec])
                u = jnp.maximum(va + vb, 0.0)
                ws = plsc.load_gather(wvv, [cvec])
                y = y + u * ws
            tv = tgtv[par][pl.ds(j * LN, LN)]
            e = jnp.abs(y + b_splat - tv)
            return acc2 + jnp.where(m, e, 0.0)

        return lax.fori_loop(0, CEC // LN, batch, acc)

    issue(0, 0)

    def chunk2(cj, acc):
        ci0 = cj * 2
        ci1 = ci0 + 1

        @pl.when(ci1 < nch)
        def _i1():
            issue(ci1, 1)

        acc = compute(ci0, 0, acc)

        @pl.when(ci0 + 2 < nch)
        def _i2():
            issue(ci0 + 2, 0)

        def _c1(a):
            return compute(ci1, 1, a)

        acc = lax.cond(ci1 < nch, _c1, lambda a: a, acc)
        return acc

    acc = lax.fori_loop(0, (nch + 1) >> 1, chunk2,
                        jnp.zeros((LN,), jnp.float32))
    accv[...] = acc
    pltpu.sync_copy(accv, out_hbm.at[pl.ds(pl.multiple_of(w * LN, 8), LN)])


_readout = pl.kernel(
    _readout_body,
    out_type=jax.ShapeDtypeStruct((NW * LN,), jnp.float32),
    mesh=_mesh,
    compiler_params=_params_nt,
    scratch_types=[
        pltpu.VMEM((CEC,), jnp.int32),
        pltpu.VMEM((CEC,), jnp.int32),
        pltpu.VMEM((CEC,), jnp.int32),
        pltpu.VMEM((CEC,), jnp.int32),
        pltpu.VMEM((CEC,), jnp.float32),
        pltpu.VMEM((CEC,), jnp.float32),
        pltpu.VMEM((CEC, D), jnp.float32),
        pltpu.VMEM((CEC, D), jnp.float32),
        pltpu.VMEM((CEC, D), jnp.float32),
        pltpu.VMEM((CEC, D), jnp.float32),
        pltpu.VMEM((D,), jnp.float32),
        pltpu.VMEM((LN,), jnp.float32),
        pltpu.VMEM((LN,), jnp.float32),
        pltpu.SemaphoreType.DMA,
        pltpu.SemaphoreType.DMA,
    ],
)


BR = SPAN


def _f32dot(a, b):
    return jnp.dot(a, b, preferred_element_type=jnp.float32)


def _tc_init_body(coord_ref, nw_ref, nb_ref, mw_ref, mb_ref, x_ref, xw_ref):
    x = _f32dot(coord_ref[...], nw_ref[...]) + nb_ref[...]
    x_ref[...] = x
    xw_ref[...] = _f32dot(x, mw_ref[...]) + mb_ref[...]


def _tc_upd_body(x_ref, agg_ref, w1_ref, w2_ref, b_ref, mw_ref, mb_ref,
                 x2_ref, xw2_ref):
    a = agg_ref[...]
    a = jnp.where(jnp.abs(a) < jnp.inf, a, 0.0)
    h = _f32dot(x_ref[...], w1_ref[...]) + _f32dot(a, w2_ref[...]) + b_ref[...]
    x2 = jnp.maximum(h, 0.0) + x_ref[...]
    x2_ref[...] = x2
    xw2_ref[...] = _f32dot(x2, mw_ref[...]) + mb_ref[...]


def _tc_fin_body(x_ref, agg_ref, w1_ref, w2_ref, b_ref, e1_ref, e2_ref,
                 eb_ref, xa_ref, xb_ref):
    a = agg_ref[...]
    a = jnp.where(jnp.abs(a) < jnp.inf, a, 0.0)
    h = _f32dot(x_ref[...], w1_ref[...]) + _f32dot(a, w2_ref[...]) + b_ref[...]
    x2 = jnp.maximum(h, 0.0) + x_ref[...]
    xa_ref[...] = _f32dot(x2, e1_ref[...])
    xb_ref[...] = _f32dot(x2, e2_ref[...]) + eb_ref[...]


def _row_spec(cols):
    return pl.BlockSpec((BR, cols), lambda i: (i, 0))


def _full_spec(r, c):
    return pl.BlockSpec((r, c), lambda i: (0, 0))


def _tc_init(coord_p, node_W, node_b, mW, mb):
    return pl.pallas_call(
        _tc_init_body,
        out_shape=(jax.ShapeDtypeStruct((NP, D), jnp.float32),
                   jax.ShapeDtypeStruct((NP, D), jnp.float32)),
        grid=(NW,),
        in_specs=[_row_spec(2), _full_spec(2, D), _full_spec(1, D),
                  _full_spec(D, D), _full_spec(1, D)],
        out_specs=(_row_spec(D), _row_spec(D)),
    )(coord_p, node_W, node_b.reshape(1, D), mW, mb.reshape(1, D))


def _tc_upd(x, agg, w1, w2, b, mw, mb):
    return pl.pallas_call(
        _tc_upd_body,
        out_shape=(jax.ShapeDtypeStruct((NP, D), jnp.float32),
                   jax.ShapeDtypeStruct((NP, D), jnp.float32)),
        grid=(NW,),
        in_specs=[_row_spec(D), _row_spec(D), _full_spec(D, D),
                  _full_spec(D, D), _full_spec(1, D), _full_spec(D, D),
                  _full_spec(1, D)],
        out_specs=(_row_spec(D), _row_spec(D)),
    )(x, agg, w1, w2, b.reshape(1, D), mw, mb.reshape(1, D))


def _tc_fin(x, agg, w1, w2, b, e1, e2, eb):
    return pl.pallas_call(
        _tc_fin_body,
        out_shape=(jax.ShapeDtypeStruct((NP, D), jnp.float32),
                   jax.ShapeDtypeStruct((NP, D), jnp.float32)),
        grid=(NW,),
        in_specs=[_row_spec(D), _row_spec(D), _full_spec(D, D),
                  _full_spec(D, D), _full_spec(1, D), _full_spec(D, D),
                  _full_spec(D, D), _full_spec(1, D)],
        out_specs=(_row_spec(D), _row_spec(D)),
    )(x, agg, w1, w2, b.reshape(1, D), e1, e2, eb.reshape(1, D))


def kernel(coord, edge_index, targets, node_W, node_b, msg_W, msg_b,
           upd_W, upd_b, edge_W, edge_b, yhat_W, yhat_b):
    src = edge_index[0]
    dst = edge_index[1]
    src_p = jnp.pad(src, (0, EPAD - E))
    dst_p = jnp.pad(dst, (0, EPAD - E))
    tgt_p = jnp.pad(targets, (0, EPAD - E))
    coord_p = jnp.pad(coord, ((0, NP - N), (0, 0)))

    pk, cnts = _partition(src, dst)
    pk2, cnt2 = _split(pk, cnts)
    x, xw = _tc_init(coord_p, node_W, node_b, msg_W[0], msg_b[0])
    for l in range(L):
        agg = _segmin(xw, pk2, cnt2)
        if l < L - 1:
            x, xw = _tc_upd(x, agg, upd_W[l][:D], upd_W[l][D:], upd_b[l],
                            msg_W[l + 1], msg_b[l + 1])
        else:
            xa, xb = _tc_fin(x, agg, upd_W[l][:D], upd_W[l][D:], upd_b[l],
                             edge_W[:D], edge_W[D:], edge_b)
    wv = yhat_W[:, 0]
    bb = jnp.full((LN,), yhat_b[0], jnp.float32)
    partials = _readout(xa, xb, src_p, dst_p, tgt_p, wv, bb)
    return jnp.sum(partials) / jnp.float32(E)


# speedup vs baseline: 1.0323x; 1.0323x over previous
import jax
import jax.numpy as jnp
from jax import lax
from jax.experimental import pallas as pl
from jax.experimental.pallas import tpu as pltpu
from jax.experimental.pallas import tpu_sc as plsc

N = 50000
E = 800000
D = 64
L = 3

NC = 2
NS = 16
NW = NC * NS
LN = 16

SPAN = 2048
SUB = 512
NSUB = SPAN // SUB
NP = NW * SPAN
STG = 4160
FL = 2048
CHA = 2000
CHA2 = 2048
CAP = E + 2 * FL
CEB = 128
DP = 128
CEC = 192
EWC = 192 * 131
EPAD = NW * EWC

_mesh = plsc.VectorSubcoreMesh(core_axis_name="c", subcore_axis_name="s")
_params = pltpu.CompilerParams(needs_layout_passes=False)
_params_nt = pltpu.CompilerParams(needs_layout_passes=False,
                                  use_tc_tiling_on_sc=False)


def _wid():
    return lax.axis_index("s") * NC + lax.axis_index("c")


def _iota():
    return lax.iota(jnp.int32, LN)


def _s0(v):
    return v[0]


def _shift_down(scr, v, k):
    scr[...] = v
    return plsc.load_gather(scr, [jnp.maximum(_iota() - k, 0)])


def _compact_store(stg, off, pk, m):
    key = jnp.where(m, _iota(), LN + _iota())
    _, spk = plsc.sort_key_val(key, pk)
    stg[pl.ds(off, LN)] = spk
    return off + _s0(plsc.all_reduce_population_count(m))


def _compact_store_vec(stg, offv, pk, m):
    key = jnp.where(m, _iota(), LN + _iota())
    _, spk = plsc.sort_key_val(key, pk)
    plsc.store_scatter(stg, [offv + _iota()], spk)
    return offv + plsc.all_reduce_population_count(m)


def _part_body(src_hbm, dst_hbm, pk_hbm, cnt_hbm, src_v, dst_v, stg_v, cnt_v):
    w = _wid()
    lo = w * SPAN

    def zi(i, c):
        stg_v[pl.ds(i * LN, LN)] = jnp.zeros((LN,), jnp.int32)
        return c

    lax.fori_loop(0, STG // LN, zi, 0)

    def chunk(ci, carry):
        offv0, flushed = carry
        pltpu.sync_copy(src_hbm.at[pl.ds(pl.multiple_of(ci * CHA, 8), CHA)],
                        src_v)
        pltpu.sync_copy(dst_hbm.at[pl.ds(pl.multiple_of(ci * CHA, 8), CHA)],
                        dst_v)

        def vstep(j, offv):
            d = dst_v[pl.ds(j * LN, LN)]
            s = src_v[pl.ds(j * LN, LN)]
            dl = d - lo
            m = (dl >= 0) & (dl < SPAN)
            pk = jnp.bitwise_or(jnp.left_shift(dl, 16), s)
            return _compact_store_vec(stg_v, offv, pk, m)

        offv = lax.fori_loop(0, CHA // LN, vstep, offv0)
        off = _s0(offv)
        do = off >= FL

        @pl.when(do)
        def _flush():
            pltpu.sync_copy(
                stg_v.at[pl.ds(0, FL)],
                pk_hbm.at[pl.ds(pl.multiple_of(w * CAP + flushed, 8), FL)])
            nt = (off - FL + LN - 1) >> 4

            def mv(i, c):
                stg_v[pl.ds(i * LN, LN)] = stg_v[pl.ds(FL + i * LN, LN)]
                return c

            lax.fori_loop(0, nt, mv, 0)

        offv = jnp.where(do, offv - FL, offv)
        flushed = jnp.where(do, flushed + FL, flushed)
        return offv, flushed

    offv, flushed = lax.fori_loop(
        0, E // CHA, chunk,
        (jnp.zeros((LN,), jnp.int32), jnp.int32(0)))
    pltpu.sync_copy(stg_v.at[pl.ds(0, FL)],
                    pk_hbm.at[pl.ds(pl.multiple_of(w * CAP + flushed, 8), FL)])
    cnt_v[...] = offv + flushed
    pltpu.sync_copy(cnt_v, cnt_hbm.at[pl.ds(pl.multiple_of(w * LN, 8), LN)])


_partition = pl.kernel(
    _part_body,
    out_type=(jax.ShapeDtypeStruct((NW * CAP,), jnp.int32),
              jax.ShapeDtypeStruct((NW * LN,), jnp.int32)),
    mesh=_mesh,
    compiler_params=_params,
    scratch_types=[
        pltpu.VMEM((CHA,), jnp.int32),
        pltpu.VMEM((CHA,), jnp.int32),
        pltpu.VMEM((STG,), jnp.int32),
        pltpu.VMEM((LN,), jnp.int32),
    ],
)


def _split_body(pk_hbm, cnt_hbm, pk2_hbm, cnt2_hbm, pkv, stg_v, cnt_v):
    w = _wid()
    iota = _iota()
    pltpu.sync_copy(cnt_hbm.at[pl.ds(pl.multiple_of(w * LN, 8), LN)], cnt_v)
    cnt = _s0(cnt_v[...])

    def zi(i, c):
        stg_v[pl.ds(i * LN, LN)] = jnp.zeros((LN,), jnp.int32)
        return c

    lax.fori_loop(0, NSUB * STG // LN, zi, 0)

    nch = (cnt + (CHA2 - 1)) >> 11

    def chunk(ci, carry):
        pltpu.sync_copy(
            pk_hbm.at[pl.ds(pl.multiple_of(w * CAP + ci * CHA2, 8), CHA2)],
            pkv)

        def vstep(j, offs):
            pk = pkv[pl.ds(j * LN, LN)]
            valid = (ci * CHA2 + j * LN + iota) < cnt
            sub = jnp.right_shift(pk, 16 + 9)
            new = []
            for p in range(NSUB):
                m = valid & (sub == p)
                new.append(_compact_store_vec(
                    stg_v.at[pl.ds(p * STG, STG)], offs[p], pk, m))
            return tuple(new)

        offvs = lax.fori_loop(0, CHA2 // LN, vstep, carry[0])
        new_offs, new_fl = [], []
        for p in range(NSUB):
            off = _s0(offvs[p])
            flushed = carry[1][p]
            do = off >= FL

            @pl.when(do)
            def _flush(p=p, flushed=flushed, off=off):
                base = (w * NSUB + p) * CAP + flushed
                pltpu.sync_copy(
                    stg_v.at[pl.ds(p * STG, FL)],
                    pk2_hbm.at[pl.ds(pl.multiple_of(base, 8), FL)])
                nt = (off - FL + LN - 1) >> 4

                def mv(i, c, p=p):
                    stg_v[pl.ds(p * STG + i * LN, LN)] = (
                        stg_v[pl.ds(p * STG + FL + i * LN, LN)])
                    return c

                lax.fori_loop(0, nt, mv, 0)

            new_offs.append(jnp.where(do, offvs[p] - FL, offvs[p]))
            new_fl.append(jnp.where(do, flushed + FL, flushed))
        return tuple(new_offs), tuple(new_fl)

    offvs, flushes = lax.fori_loop(
        0, nch, chunk,
        ((jnp.zeros((LN,), jnp.int32),) * NSUB, (jnp.int32(0),) * NSUB))
    for p in range(NSUB):
        flushed = flushes[p]
        base = (w * NSUB + p) * CAP + flushed
        pltpu.sync_copy(stg_v.at[pl.ds(p * STG, FL)],
                        pk2_hbm.at[pl.ds(pl.multiple_of(base, 8), FL)])
        cnt_v[...] = offvs[p] + flushed
        pltpu.sync_copy(
            cnt_v,
            cnt2_hbm.at[pl.ds(pl.multiple_of((w * NSUB + p) * LN, 8), LN)])


_split = pl.kernel(
    _split_body,
    out_type=(jax.ShapeDtypeStruct((NW * NSUB * CAP,), jnp.int32),
              jax.ShapeDtypeStruct((NW * NSUB * LN,), jnp.int32)),
    mesh=_mesh,
    compiler_params=_params,
    scratch_types=[
        pltpu.VMEM((CHA2,), jnp.int32),
        pltpu.VMEM((NSUB * STG,), jnp.int32),
        pltpu.VMEM((LN,), jnp.int32),
    ],
)


def _segmin_body(xw_hbm, pk2_hbm, cnt2_hbm, agg_hbm,
                 pkv0, pkv1, idxv0, idxv1, dlv0, dlv1, rows0, rows1, agg,
                 cntv, sem0, sem1):
    w = _wid()
    iota = _iota()
    inf_v = jnp.full((LN,), jnp.inf, jnp.float32)
    pkv = (pkv0, pkv1)
    idxv = (idxv0, idxv1)
    dlv = (dlv0, dlv1)
    rows = (rows0, rows1)
    sem = (sem0, sem1)

    for p in range(NSUB):
        pltpu.sync_copy(
            cnt2_hbm.at[pl.ds(pl.multiple_of((w * NSUB + p) * LN, 8), LN)],
            cntv)
        cnt = _s0(cntv[...])

        def zrow(r, c):
            for cc in range(D // LN):
                agg[r, pl.ds(cc * LN, LN)] = inf_v
            return c

        lax.fori_loop(0, SUB, zrow, 0)

        nch = (cnt + (CEB - 1)) >> 7
        lbase = (w * NSUB + p) * CAP

        def issue(ci, par, lbase=lbase):
            pltpu.sync_copy(
                pk2_hbm.at[pl.ds(pl.multiple_of(lbase + ci * CEB, 8), CEB)],
                pkv[par])
            for j in range(CEB // LN):
                pk = pkv[par][pl.ds(j * LN, LN)]
                idxv[par][pl.ds(j * LN, LN)] = pk & 0xFFFF
                dlv[par][pl.ds(j * LN, LN)] = (
                    jnp.right_shift(pk, 16) & (SUB - 1))
            pltpu.async_copy(xw_hbm.at[idxv[par]], rows[par], sem[par])

        def compute(ci, par, cnt=cnt):
            pltpu.make_async_copy(xw_hbm.at[idxv[par]], rows[par],
                                  sem[par]).wait()

            def batch(j, c1):
                dl = dlv[par][pl.ds(j * LN, LN)]
                valid = (ci * CEB + j * LN + iota) < cnt
                rowi = j * LN + iota
                for k in range(D):
                    cvec = (iota + k) & (D - 1)
                    rv = plsc.load_gather(rows[par], [rowi, cvec])
                    av = plsc.load_gather(agg, [dl, cvec])
                    plsc.store_scatter(agg, [dl, cvec],
                                       jnp.minimum(av, rv), mask=valid)
                return c1

            lax.fori_loop(0, CEB // LN, batch, 0)

        @pl.when(nch > 0)
        def _prime():
            issue(0, 0)

        def chunk2(cj, c0):
            ci0 = cj * 2
            ci1 = ci0 + 1

            @pl.when(ci1 < nch)
            def _i1():
                issue(ci1, 1)

            compute(ci0, 0)

            @pl.when(ci0 + 2 < nch)
            def _i2():
                issue(ci0 + 2, 0)

            @pl.when(ci1 < nch)
            def _c1():
                compute(ci1, 1)

            return c0

        lax.fori_loop(0, (nch + 1) >> 1, chunk2, 0)
        pltpu.sync_copy(agg, agg_hbm.at[pl.ds(w * SPAN + p * SUB, SUB)])


_segmin = pl.kernel(
    _segmin_body,
    out_type=jax.ShapeDtypeStruct((NP, D), jnp.float32),
    mesh=_mesh,
    compiler_params=_params_nt,
    scratch_types=[
        pltpu.VMEM((CEB,), jnp.int32),
        pltpu.VMEM((CEB,), jnp.int32),
        pltpu.VMEM((CEB,), jnp.int32),
        pltpu.VMEM((CEB,), jnp.int32),
        pltpu.VMEM((CEB,), jnp.int32),
        pltpu.VMEM((CEB,), jnp.int32),
        pltpu.VMEM((CEB, D), jnp.float32),
        pltpu.VMEM((CEB, D), jnp.float32),
        pltpu.VMEM((SUB, D), jnp.float32),
        pltpu.VMEM((LN,), jnp.int32),
        pltpu.SemaphoreType.DMA,
        pltpu.SemaphoreType.DMA,
    ],
)


def _readout_body(xa_hbm, xb_hbm, src_hbm, dst_hbm, tgt_hbm, wv_hbm, bb_hbm,
                  out_hbm, srcv0, srcv1, dstv0, dstv1, tgtv0, tgtv1,
                  rowsa0, rowsa1, rowsb0, rowsb1, wvv, bbv, accv,
                  sem0, sem1):
    w = _wid()
    iota = _iota()
    pltpu.sync_copy(wv_hbm, wvv)
    pltpu.sync_copy(bb_hbm, bbv)
    b_splat = bbv[...]
    base = w * EWC
    srcv = (srcv0, srcv1)
    dstv = (dstv0, dstv1)
    tgtv = (tgtv0, tgtv1)
    rowsa = (rowsa0, rowsa1)
    rowsb = (rowsb0, rowsb1)
    sem = (sem0, sem1)
    nch = EWC // CEC

    def issue(ci, par):
        off = base + ci * CEC
        pltpu.sync_copy(src_hbm.at[pl.ds(pl.multiple_of(off, 8), CEC)],
                        srcv[par])
        pltpu.sync_copy(dst_hbm.at[pl.ds(pl.multiple_of(off, 8), CEC)],
                        dstv[par])
        pltpu.sync_copy(tgt_hbm.at[pl.ds(pl.multiple_of(off, 8), CEC)],
                        tgtv[par])
        pltpu.async_copy(xa_hbm.at[srcv[par].at[pl.ds(0, 96)]],
                         rowsa[par].at[pl.ds(0, 96)], sem[par])
        pltpu.async_copy(xa_hbm.at[srcv[par].at[pl.ds(96, 96)]],
                         rowsa[par].at[pl.ds(96, 96)], sem[par])
        pltpu.async_copy(xb_hbm.at[dstv[par].at[pl.ds(0, 96)]],
                         rowsb[par].at[pl.ds(0, 96)], sem[par])
        pltpu.async_copy(xb_hbm.at[dstv[par].at[pl.ds(96, 96)]],
                         rowsb[par].at[pl.ds(96, 96)], sem[par])

    def compute(ci, par, acc):
        off = base + ci * CEC
        pltpu.make_async_copy(xa_hbm.at[srcv[par]], rowsa[par],
                              sem[par]).wait()
        pltpu.make_async_copy(xb_hbm.at[dstv[par]], rowsb[par],
                              sem[par]).wait()

        def batch(j, acc2):
            rowi = j * LN + iota
            m = (off + rowi) < E
            y = jnp.zeros((LN,), jnp.float32)
            for k in range(D):
                cvec = (iota + k) & (D - 1)
                va = plsc.load_gather(rowsa[par], [rowi, cvec])
                vb = plsc.load_gather(rowsb[par], [rowi, cvec])
                u = jnp.maximum(va + vb, 0.0)
                ws = plsc.load_gather(wvv, [cvec])
                y = y + u * ws
            tv = tgtv[par][pl.ds(j * LN, LN)]
            e = jnp.abs(y + b_splat - tv)
            return acc2 + jnp.where(m, e, 0.0)

        return lax.fori_loop(0, CEC // LN, batch, acc)

    issue(0, 0)

    def chunk2(cj, acc):
        ci0 = cj * 2
        ci1 = ci0 + 1

        @pl.when(ci1 < nch)
        def _i1():
            issue(ci1, 1)

        acc = compute(ci0, 0, acc)

        @pl.when(ci0 + 2 < nch)
        def _i2():
            issue(ci0 + 2, 0)

        def _c1(a):
            return compute(ci1, 1, a)

        acc = lax.cond(ci1 < nch, _c1, lambda a: a, acc)
        return acc

    acc = lax.fori_loop(0, (nch + 1) >> 1, chunk2,
                        jnp.zeros((LN,), jnp.float32))
    accv[...] = acc
    pltpu.sync_copy(accv, out_hbm.at[pl.ds(pl.multiple_of(w * LN, 8), LN)])


_readout = pl.kernel(
    _readout_body,
    out_type=jax.ShapeDtypeStruct((NW * LN,), jnp.float32),
    mesh=_mesh,
    compiler_params=_params_nt,
    scratch_types=[
        pltpu.VMEM((CEC,), jnp.int32),
        pltpu.VMEM((CEC,), jnp.int32),
        pltpu.VMEM((CEC,), jnp.int32),
        pltpu.VMEM((CEC,), jnp.int32),
        pltpu.VMEM((CEC,), jnp.float32),
        pltpu.VMEM((CEC,), jnp.float32),
        pltpu.VMEM((CEC, D), jnp.float32),
        pltpu.VMEM((CEC, D), jnp.float32),
        pltpu.VMEM((CEC, D), jnp.float32),
        pltpu.VMEM((CEC, D), jnp.float32),
        pltpu.VMEM((D,), jnp.float32),
        pltpu.VMEM((LN,), jnp.float32),
        pltpu.VMEM((LN,), jnp.float32),
        pltpu.SemaphoreType.DMA,
        pltpu.SemaphoreType.DMA,
    ],
)


BR = SPAN


def _f32dot(a, b):
    return jnp.dot(a, b, preferred_element_type=jnp.float32)


def _tc_init_body(coord_ref, nw_ref, nb_ref, mw_ref, mb_ref, x_ref, xw_ref):
    x = _f32dot(coord_ref[...], nw_ref[...]) + nb_ref[...]
    x_ref[...] = x
    xw_ref[...] = _f32dot(x, mw_ref[...]) + mb_ref[...]


def _tc_upd_body(x_ref, agg_ref, w1_ref, w2_ref, b_ref, mw_ref, mb_ref,
                 x2_ref, xw2_ref):
    a = agg_ref[...]
    a = jnp.where(jnp.abs(a) < jnp.inf, a, 0.0)
    h = _f32dot(x_ref[...], w1_ref[...]) + _f32dot(a, w2_ref[...]) + b_ref[...]
    x2 = jnp.maximum(h, 0.0) + x_ref[...]
    x2_ref[...] = x2
    xw2_ref[...] = _f32dot(x2, mw_ref[...]) + mb_ref[...]


def _tc_fin_body(x_ref, agg_ref, w1_ref, w2_ref, b_ref, e1_ref, e2_ref,
                 eb_ref, xa_ref, xb_ref):
    a = agg_ref[...]
    a = jnp.where(jnp.abs(a) < jnp.inf, a, 0.0)
    h = _f32dot(x_ref[...], w1_ref[...]) + _f32dot(a, w2_ref[...]) + b_ref[...]
    x2 = jnp.maximum(h, 0.0) + x_ref[...]
    xa_ref[...] = _f32dot(x2, e1_ref[...])
    xb_ref[...] = _f32dot(x2, e2_ref[...]) + eb_ref[...]


def _row_spec(cols):
    return pl.BlockSpec((BR, cols), lambda i: (i, 0))


def _full_spec(r, c):
    return pl.BlockSpec((r, c), lambda i: (0, 0))


def _tc_init(coord_p, node_W, node_b, mW, mb):
    return pl.pallas_call(
        _tc_init_body,
        out_shape=(jax.ShapeDtypeStruct((NP, D), jnp.float32),
                   jax.ShapeDtypeStruct((NP, D), jnp.float32)),
        grid=(NW,),
        in_specs=[_row_spec(2), _full_spec(2, D), _full_spec(1, D),
                  _full_spec(D, D), _full_spec(1, D)],
        out_specs=(_row_spec(D), _row_spec(D)),
    )(coord_p, node_W, node_b.reshape(1, D), mW, mb.reshape(1, D))


def _tc_upd(x, agg, w1, w2, b, mw, mb):
    return pl.pallas_call(
        _tc_upd_body,
        out_shape=(jax.ShapeDtypeStruct((NP, D), jnp.float32),
                   jax.ShapeDtypeStruct((NP, D), jnp.float32)),
        grid=(NW,),
        in_specs=[_row_spec(D), _row_spec(D), _full_spec(D, D),
                  _full_spec(D, D), _full_spec(1, D), _full_spec(D, D),
                  _full_spec(1, D)],
        out_specs=(_row_spec(D), _row_spec(D)),
    )(x, agg, w1, w2, b.reshape(1, D), mw, mb.reshape(1, D))


def _tc_fin(x, agg, w1, w2, b, e1, e2, eb):
    return pl.pallas_call(
        _tc_fin_body,
        out_shape=(jax.ShapeDtypeStruct((NP, D), jnp.float32),
                   jax.ShapeDtypeStruct((NP, D), jnp.float32)),
        grid=(NW,),
        in_specs=[_row_spec(D), _row_spec(D), _full_spec(D, D),
                  _full_spec(D, D), _full_spec(1, D), _full_spec(D, D),
                  _full_spec(D, D), _full_spec(1, D)],
        out_specs=(_row_spec(D), _row_spec(D)),
    )(x, agg, w1, w2, b.reshape(1, D), e1, e2, eb.reshape(1, D))


def kernel(coord, edge_index, targets, node_W, node_b, msg_W, msg_b,
           upd_W, upd_b, edge_W, edge_b, yhat_W, yhat_b):
    src = edge_index[0]
    dst = edge_index[1]
    src_p = jnp.pad(src, (0, EPAD - E))
    dst_p = jnp.pad(dst, (0, EPAD - E))
    tgt_p = jnp.pad(targets, (0, EPAD - E))
    coord_p = jnp.pad(coord, ((0, NP - N), (0, 0)))

    pk, cnts = _partition(src, dst)
    pk2, cnt2 = _split(pk, cnts)
    x, xw = _tc_init(coord_p, node_W, node_b, msg_W[0], msg_b[0])
    for l in range(L):
        agg = _segmin(xw, pk2, cnt2)
        if l < L - 1:
            x, xw = _tc_upd(x, agg, upd_W[l][:D], upd_W[l][D:], upd_b[l],
                            msg_W[l + 1], msg_b[l + 1])
        else:
            xa, xb = _tc_fin(x, agg, upd_W[l][:D], upd_W[l][D:], upd_b[l],
                             edge_W[:D], edge_W[D:], edge_b)
    wv = yhat_W[:, 0]
    bb = jnp.full((LN,), yhat_b[0], jnp.float32)
    partials = _readout(xa, xb, src_p, dst_p, tgt_p, wv, bb)
    return jnp.sum(partials) / jnp.float32(E)
